# baseline (speedup 1.0000x reference)
import functools

import jax
import jax.numpy as jnp
import numpy as np
from jax import lax
from jax.experimental import pallas as pl
from jax.experimental.pallas import tpu as pltpu
from jax.experimental.pallas import tpu_sc as plsc

B = 16384
F = 26
D = 16
NC, NS = 2, 16
NW = NC * NS
SPW = B // NW
CH = 64
NCH = SPW // CH
NR = CH * F
KI = NR // 128
INV_F = 1.0 / F
TBLK = 2048

_OFFS = np.arange(F, dtype=np.int32) * 100000

_mesh = plsc.VectorSubcoreMesh(core_axis_name="c", subcore_axis_name="s")


@functools.partial(
    pl.kernel,
    mesh=_mesh,
    compiler_params=pltpu.CompilerParams(use_tc_tiling_on_sc=False),
    out_type=(
        jax.ShapeDtypeStruct((B, D), jnp.float32),
        jax.ShapeDtypeStruct((B,), jnp.float32),
    ),
    scratch_types=[
        pltpu.VMEM((2, KI, 128), jnp.int32),
        pltpu.VMEM((2, NR, D), jnp.float32),
        pltpu.VMEM((2, NR), jnp.float32),
        pltpu.VMEM((2, CH, D), jnp.float32),
        pltpu.VMEM((2, CH), jnp.float32),
        pltpu.SemaphoreType.DMA,
        pltpu.SemaphoreType.DMA,
        pltpu.SemaphoreType.DMA,
    ],
)
def _sc_bag(idx_hbm, emb_hbm, lin_hbm, oemb_hbm, olin_hbm,
            idx_v, rows_v, linv_v, oemb_v, olin_v, sem0, sem1, osem):
    wid = lax.axis_index("s") * NC + lax.axis_index("c")
    base = wid * SPW
    gsems = (sem0, sem1)

    def fire(j):
        b = j % 2
        pltpu.sync_copy(idx_hbm.at[wid, j], idx_v.at[b])
        cps = []
        for k in range(KI):
            cps.append(pltpu.async_copy(
                emb_hbm.at[idx_v.at[b, k]],
                rows_v.at[b, pl.ds(k * 128, 128)], gsems[b]))
            cps.append(pltpu.async_copy(
                lin_hbm.at[idx_v.at[b, k]],
                linv_v.at[b, pl.ds(k * 128, 128)], gsems[b]))
        return cps

    pending = fire(0)
    out_cps = []
    for j in range(NCH):
        b = j % 2
        nxt = fire(j + 1) if j + 1 < NCH else []
        for c in pending:
            c.wait()
        pending = nxt
        if j >= 2:
            for c in out_cps[j - 2]:
                c.wait()

        def red(s, carry):
            a0 = rows_v[b, s, :]
            a1 = rows_v[b, CH + s, :]
            a2 = rows_v[b, 2 * CH + s, :]
            a3 = rows_v[b, 3 * CH + s, :]
            for f in range(4, F):
                r = rows_v[b, f * CH + s, :]
                if f % 4 == 0:
                    a0 = a0 + r
                elif f % 4 == 1:
                    a1 = a1 + r
                elif f % 4 == 2:
                    a2 = a2 + r
                else:
                    a3 = a3 + r
            oemb_v[b, s, :] = (a0 + a1) + (a2 + a3)
            return carry
        lax.fori_loop(0, CH, red, 0, unroll=2)

        for g in range(CH // 16):
            l0 = linv_v[b, pl.ds(g * 16, 16)]
            l1 = linv_v[b, pl.ds(CH + g * 16, 16)]
            l2 = linv_v[b, pl.ds(2 * CH + g * 16, 16)]
            l3 = linv_v[b, pl.ds(3 * CH + g * 16, 16)]
            for f in range(4, F):
                r = linv_v[b, pl.ds(f * CH + g * 16, 16)]
                if f % 4 == 0:
                    l0 = l0 + r
                elif f % 4 == 1:
                    l1 = l1 + r
                elif f % 4 == 2:
                    l2 = l2 + r
                else:
                    l3 = l3 + r
            olin_v[b, pl.ds(g * 16, 16)] = (l0 + l1) + (l2 + l3)

        s0 = base + j * CH
        out_cps.append([
            pltpu.async_copy(oemb_v.at[b], oemb_hbm.at[pl.ds(s0, CH)], osem),
            pltpu.async_copy(olin_v.at[b], olin_hbm.at[pl.ds(s0, CH)], osem),
        ])
    for cs in out_cps[-2:]:
        for c in cs:
            c.wait()


def _tc_head(esum_ref, lsum_ref, w1_ref, b1_ref, w2_ref, b2_ref, w3_ref,
             bias_ref, out_ref):
    e = esum_ref[...] * INV_F
    s = jnp.sum(e, axis=1, keepdims=True)
    sq = jnp.sum(e * e, axis=1, keepdims=True)
    fm = 0.5 * (s * s - sq)
    h = jnp.maximum(
        jnp.dot(e, w1_ref[...], preferred_element_type=jnp.float32)
        + b1_ref[...], 0.0)
    h = jnp.maximum(
        jnp.dot(h, w2_ref[...], preferred_element_type=jnp.float32)
        + b2_ref[...], 0.0)
    mlp = jnp.sum(h * w3_ref[...], axis=1, keepdims=True)
    z = lsum_ref[...] * INV_F + fm + mlp + bias_ref[0]
    out_ref[...] = 1.0 / (1.0 + jnp.exp(-z))


_tc_call = pl.pallas_call(
    _tc_head,
    grid=(B // TBLK,),
    in_specs=[
        pl.BlockSpec((TBLK, D), lambda i: (i, 0)),
        pl.BlockSpec((TBLK, 1), lambda i: (i, 0)),
        pl.BlockSpec((D, 128), lambda i: (0, 0)),
        pl.BlockSpec((1, 128), lambda i: (0, 0)),
        pl.BlockSpec((128, 64), lambda i: (0, 0)),
        pl.BlockSpec((1, 64), lambda i: (0, 0)),
        pl.BlockSpec((1, 64), lambda i: (0, 0)),
        pl.BlockSpec(memory_space=pltpu.SMEM),
    ],
    out_specs=pl.BlockSpec((TBLK, 1), lambda i: (i, 0)),
    out_shape=jax.ShapeDtypeStruct((B, 1), jnp.float32),
)


def kernel(x, emb_table, lin_table, lin_bias, W1, b1, W2, b2, W3, b3):
    idx = (x + jnp.asarray(_OFFS)[None, :]).reshape(NW, NCH, CH, F)
    idx = idx.transpose(0, 1, 3, 2).reshape(NW, NCH, KI, 128)
    esum, lsum = _sc_bag(idx, emb_table, lin_table.T.reshape(-1))
    bias = (lin_bias + b3).reshape(1)
    out = _tc_call(esum, lsum.reshape(B, 1), W1, b1.reshape(1, 128),
                   W2, b2.reshape(1, 64), W3.reshape(1, 64), bias)
    return out.reshape(B)
